# baseline (speedup 1.0000x reference)
import functools

import jax
import jax.numpy as jnp
from jax import lax
from jax.experimental import pallas as pl
from jax.experimental.pallas import tpu as pltpu
from jax.experimental.pallas import tpu_sc as plsc

NC = 2
NS = 16
NW = NC * NS
CHUNK = 128


@functools.lru_cache(maxsize=None)
def _make_kernel(B, D):
    b_per_w = B // NW
    n_chunks = b_per_w // CHUNK
    mesh = plsc.VectorSubcoreMesh(core_axis_name="c", subcore_axis_name="s")

    @functools.partial(
        pl.kernel,
        mesh=mesh,
        out_type=jax.ShapeDtypeStruct((B, 2 * D), jnp.bfloat16),
        scratch_types=[
            pltpu.VMEM((b_per_w,), jnp.int32),
            pltpu.VMEM((b_per_w,), jnp.int32),
            pltpu.VMEM((b_per_w, D), jnp.bfloat16),
            pltpu.VMEM((b_per_w, D), jnp.bfloat16),
            pltpu.SemaphoreType.DMA,
            pltpu.SemaphoreType.DMA,
        ],
        compiler_params=pltpu.CompilerParams(use_tc_tiling_on_sc=False),
    )
    def k(user_hbm, item_hbm, uidx_hbm, iidx_hbm, out_hbm,
          uidx_v, iidx_v, urows_v, irows_v, usem, isem):
        wid = lax.axis_index("s") * NC + lax.axis_index("c")
        base = wid * b_per_w
        pltpu.sync_copy(uidx_hbm.at[pl.ds(base, b_per_w)], uidx_v)
        pltpu.sync_copy(iidx_hbm.at[pl.ds(base, b_per_w)], iidx_v)
        copies = []
        for j in range(n_chunks):
            uidx = uidx_v.at[pl.ds(j * CHUNK, CHUNK)]
            iidx = iidx_v.at[pl.ds(j * CHUNK, CHUNK)]
            copies.append(pltpu.async_copy(
                user_hbm.at[uidx],
                urows_v.at[pl.ds(j * CHUNK, CHUNK)], usem))
            copies.append(pltpu.async_copy(
                item_hbm.at[iidx],
                irows_v.at[pl.ds(j * CHUNK, CHUNK)], isem))
        for c in copies:
            c.wait()
        pltpu.sync_copy(urows_v, out_hbm.at[pl.ds(base, b_per_w), pl.ds(0, D)])
        pltpu.sync_copy(irows_v, out_hbm.at[pl.ds(base, b_per_w), pl.ds(D, D)])

    return k


def kernel(user_embedding, item_embedding, user_idx, item_idx):
    B = user_idx.shape[0]
    D = user_embedding.shape[1]
    out16 = _make_kernel(B, D)(
        user_embedding.astype(jnp.bfloat16),
        item_embedding.astype(jnp.bfloat16),
        user_idx.astype(jnp.int32), item_idx.astype(jnp.int32))
    return out16.astype(jnp.float32)


# speedup vs baseline: 1.1709x; 1.1709x over previous
import functools

import jax
import jax.numpy as jnp
from jax import lax
from jax.experimental import pallas as pl
from jax.experimental.pallas import tpu as pltpu
from jax.experimental.pallas import tpu_sc as plsc

NC = 2
NS = 16
NW = NC * NS
CHUNK = 128


@functools.lru_cache(maxsize=None)
def _make_kernel(B, D):
    b_per_w = B // NW
    n_chunks = b_per_w // CHUNK
    mesh = plsc.VectorSubcoreMesh(core_axis_name="c", subcore_axis_name="s")

    @functools.partial(
        pl.kernel,
        mesh=mesh,
        out_type=jax.ShapeDtypeStruct((B, 2 * D), jnp.float32),
        scratch_types=[
            pltpu.VMEM((b_per_w,), jnp.int32),
            pltpu.VMEM((b_per_w,), jnp.int32),
            pltpu.VMEM((b_per_w, D), jnp.float32),
            pltpu.VMEM((b_per_w, D), jnp.float32),
            pltpu.SemaphoreType.DMA,
            pltpu.SemaphoreType.DMA,
        ],
        compiler_params=pltpu.CompilerParams(use_tc_tiling_on_sc=False),
    )
    def k(user_hbm, item_hbm, uidx_hbm, iidx_hbm, out_hbm,
          uidx_v, iidx_v, urows_v, irows_v, usem, isem):
        wid = lax.axis_index("s") * NC + lax.axis_index("c")
        base = wid * b_per_w
        pltpu.sync_copy(uidx_hbm.at[pl.ds(base, b_per_w)], uidx_v)
        pltpu.sync_copy(iidx_hbm.at[pl.ds(base, b_per_w)], iidx_v)
        copies = []
        for j in range(n_chunks):
            uidx = uidx_v.at[pl.ds(j * CHUNK, CHUNK)]
            iidx = iidx_v.at[pl.ds(j * CHUNK, CHUNK)]
            copies.append(pltpu.async_copy(
                user_hbm.at[uidx],
                urows_v.at[pl.ds(j * CHUNK, CHUNK)], usem))
            copies.append(pltpu.async_copy(
                item_hbm.at[iidx],
                irows_v.at[pl.ds(j * CHUNK, CHUNK)], isem))
        for c in copies:
            c.wait()
        pltpu.sync_copy(urows_v, out_hbm.at[pl.ds(base, b_per_w), pl.ds(0, D)])
        pltpu.sync_copy(irows_v, out_hbm.at[pl.ds(base, b_per_w), pl.ds(D, D)])

    return k


def kernel(user_embedding, item_embedding, user_idx, item_idx):
    B = user_idx.shape[0]
    D = user_embedding.shape[1]
    return _make_kernel(B, D)(
        user_embedding, item_embedding,
        user_idx.astype(jnp.int32), item_idx.astype(jnp.int32))
